# baseline (speedup 1.0000x reference)
import functools

import jax
import jax.numpy as jnp
from jax import lax
from jax.experimental import pallas as pl
from jax.experimental.pallas import tpu as pltpu
from jax.experimental.pallas import tpu_sc as plsc

D_MODEL = 2048
B_TOTAL = 16384

_info = plsc.get_sparse_core_info()
_NC = _info.num_cores
_NS = _info.num_subcores
_NW = _NC * _NS
_BPW = B_TOTAL // _NW
_K = 4
_NBUF = 12
_LOOK = 6
_NCHUNK = _BPW // _K
assert B_TOTAL % _NW == 0 and _BPW % _K == 0

_mesh = plsc.VectorSubcoreMesh(core_axis_name="c", subcore_axis_name="s")


@functools.partial(
    pl.kernel,
    mesh=_mesh,
    out_type=jax.ShapeDtypeStruct((B_TOTAL, D_MODEL), jnp.float32),
    scratch_types=(
        [pltpu.VMEM((_NCHUNK, _K), jnp.int32)]
        + [pltpu.VMEM((_K, D_MODEL), jnp.float32) for _ in range(_NBUF)]
        + [pltpu.SemaphoreType.DMA for _ in range(2 * _NBUF)]
    ),
)
def _embed_gather(ids_hbm, table_hbm, out_hbm, idx_v, *bufs_and_sems):
    bufs = bufs_and_sems[:_NBUF]
    gsem = bufs_and_sems[_NBUF:2 * _NBUF]
    ssem = bufs_and_sems[2 * _NBUF:]
    wid = lax.axis_index("s") * _NC + lax.axis_index("c")
    base = wid * _BPW
    pltpu.sync_copy(ids_hbm.at[wid], idx_v)

    def start_gather(c):
        b = c % _NBUF
        return pltpu.async_copy(table_hbm.at[idx_v.at[c]], bufs[b], gsem[b])

    def start_store(c):
        b = c % _NBUF
        return pltpu.async_copy(
            bufs[b], out_hbm.at[pl.ds(base + c * _K, _K)], ssem[b])

    pend_g = {c: start_gather(c) for c in range(_LOOK)}
    pend_s = {}
    for c in range(_NCHUNK):
        pend_g.pop(c).wait()
        pend_s[c] = start_store(c)
        nc = c + _LOOK
        if nc < _NCHUNK:
            prev = nc - _NBUF
            if prev in pend_s:
                pend_s.pop(prev).wait()
            pend_g[nc] = start_gather(nc)
    for c in sorted(pend_s):
        pend_s.pop(c).wait()


def kernel(input_ids, labels, W):
    batch, seq = input_ids.shape
    ids = input_ids.astype(jnp.int32).reshape(_NW, _NCHUNK, _K)
    out = _embed_gather(ids, W)
    return (out.reshape(batch, seq, D_MODEL), labels)


# speedup vs baseline: 1.0221x; 1.0221x over previous
import functools

import jax
import jax.numpy as jnp
from jax import lax
from jax.experimental import pallas as pl
from jax.experimental.pallas import tpu as pltpu
from jax.experimental.pallas import tpu_sc as plsc

D_MODEL = 2048
B_TOTAL = 16384

_info = plsc.get_sparse_core_info()
_NC = _info.num_cores
_NS = _info.num_subcores
_NW = _NC * _NS
_BPW = B_TOTAL // _NW
_K = 8
_NBUF = 6
_LOOK = 3
_NCHUNK = _BPW // _K
assert B_TOTAL % _NW == 0 and _BPW % _K == 0

_mesh = plsc.VectorSubcoreMesh(core_axis_name="c", subcore_axis_name="s")


@functools.partial(
    pl.kernel,
    mesh=_mesh,
    out_type=jax.ShapeDtypeStruct((B_TOTAL, D_MODEL), jnp.float32),
    scratch_types=(
        [pltpu.VMEM((_NCHUNK, _K), jnp.int32)]
        + [pltpu.VMEM((_K, D_MODEL), jnp.float32) for _ in range(_NBUF)]
        + [pltpu.SemaphoreType.DMA for _ in range(2 * _NBUF)]
    ),
)
def _embed_gather(ids_hbm, table_hbm, out_hbm, idx_v, *bufs_and_sems):
    bufs = bufs_and_sems[:_NBUF]
    gsem = bufs_and_sems[_NBUF:2 * _NBUF]
    ssem = bufs_and_sems[2 * _NBUF:]
    wid = lax.axis_index("s") * _NC + lax.axis_index("c")
    base = wid * _BPW
    pltpu.sync_copy(ids_hbm.at[wid], idx_v)

    def start_gather(c):
        b = c % _NBUF
        return pltpu.async_copy(table_hbm.at[idx_v.at[c]], bufs[b], gsem[b])

    def start_store(c):
        b = c % _NBUF
        return pltpu.async_copy(
            bufs[b], out_hbm.at[pl.ds(base + c * _K, _K)], ssem[b])

    pend_g = {c: start_gather(c) for c in range(_LOOK)}
    pend_s = {}
    for c in range(_NCHUNK):
        pend_g.pop(c).wait()
        pend_s[c] = start_store(c)
        nc = c + _LOOK
        if nc < _NCHUNK:
            prev = nc - _NBUF
            if prev in pend_s:
                pend_s.pop(prev).wait()
            pend_g[nc] = start_gather(nc)
    for c in sorted(pend_s):
        pend_s.pop(c).wait()


def kernel(input_ids, labels, W):
    batch, seq = input_ids.shape
    ids = input_ids.astype(jnp.int32).reshape(_NW, _NCHUNK, _K)
    out = _embed_gather(ids, W)
    return (out.reshape(batch, seq, D_MODEL), labels)
